# baseline (speedup 1.0000x reference)
import jax
import jax.numpy as jnp
import numpy as np
from jax.experimental import pallas as pl
from jax.experimental.shard_map import shard_map
from jax.sharding import Mesh, PartitionSpec as P

N = 1024
B = 2
HID = 1024
LAT = 512
NHEADS = 4
NHID = 64
ALPHA = 0.2
_NEG_LOG2E = -1.4426950408889634


def _elu(v):
    return jnp.where(v > 0, v, jnp.exp(v) - 1.0)


def _deepim_body(x_ref, adj_ref, w1_ref, b1_ref, w2_ref, b2_ref, w3_ref, b3_ref,
                 d1_ref, db1_ref, d2_ref, db2_ref, d3_ref, db3_ref, d4_ref, db4_ref,
                 gw_ref, ga_ref, ow_ref, oa_ref, xhat_ref, yhat_ref):
    x = x_ref[...]
    adj = adj_ref[...]

    def _mm(a, w_ref):
        return jnp.dot(a, w_ref[...], preferred_element_type=jnp.float32)

    h = jax.nn.relu(_mm(x, w1_ref) + b1_ref[...])
    h = jax.nn.relu(_mm(h, w2_ref) + b2_ref[...])
    h = jax.nn.relu(_mm(h, w2_ref) + b2_ref[...])
    z = _mm(h, w3_ref) + b3_ref[...]
    h = jax.nn.relu(_mm(z, d1_ref) + db1_ref[...])
    h = jax.nn.relu(_mm(h, d2_ref) + db2_ref[...])
    h = jax.nn.relu(_mm(h, d3_ref) + db3_ref[...])
    x_hat = jax.nn.sigmoid(_mm(h, d4_ref) + db4_ref[...])
    xhat_ref[...] = x_hat

    gw = gw_ref[...]
    ga = ga_ref[...]
    ow = ow_ref[...]
    oa = oa_ref[...]

    for b in range(x.shape[0]):
        xn_row = x_hat[b:b + 1, :]
        xn_col = jnp.transpose(xn_row)

        blocks = []
        for i in range(NHEADS):
            gw_i = gw[i:i + 1, :]
            c1 = jnp.sum(gw_i * ga[i:i + 1, :NHID], axis=1, keepdims=True)
            c2 = jnp.sum(gw_i * ga[i:i + 1, NHID:], axis=1, keepdims=True)
            m1 = xn_col * (c1 * _NEG_LOG2E)
            n1 = xn_row * (c2 * _NEG_LOG2E)
            sp = m1 + n1
            e = jnp.exp2(jnp.minimum(sp, ALPHA * sp)) * adj
            den = jnp.sum(e, axis=1, keepdims=True)
            num = jnp.sum(e * xn_row, axis=1, keepdims=True)
            p = num / den
            blocks.append(_elu(_elu(p * gw_i)))
        xh = jnp.concatenate(blocks, axis=1)

        h2 = jnp.dot(xh, ow, preferred_element_type=jnp.float32)
        h2_row = jnp.transpose(h2)
        m1 = h2 * (oa[0:1, 0:1] * _NEG_LOG2E)
        n1 = h2_row * (oa[0:1, 1:2] * _NEG_LOG2E)
        sp = m1 + n1
        e2 = jnp.exp2(jnp.minimum(sp, ALPHA * sp)) * adj
        den2 = jnp.sum(e2, axis=1, keepdims=True)
        num2 = jnp.sum(e2 * h2_row, axis=1, keepdims=True)
        y = _elu(num2 / den2)
        yhat_ref[b:b + 1, :] = jnp.transpose(y)


def _run(xb, *rest):
    x_hat, y_hat = pl.pallas_call(
        _deepim_body,
        out_shape=(
            jax.ShapeDtypeStruct((xb.shape[0], N), jnp.float32),
            jax.ShapeDtypeStruct((xb.shape[0], N), jnp.float32),
        ),
    )(xb, *rest)
    return x_hat, y_hat


def kernel(x, adj, enc_w1, enc_b1, enc_w2, enc_b2, enc_w3, enc_b3,
           dec_w1, dec_b1, dec_w2, dec_b2, dec_w3, dec_b3, dec_w4, dec_b4,
           gat_W, gat_a, out_W, out_a):
    rest = (
        adj,
        enc_w1, enc_b1.reshape(1, HID), enc_w2, enc_b2.reshape(1, HID),
        enc_w3, enc_b3.reshape(1, LAT),
        dec_w1, dec_b1.reshape(1, LAT), dec_w2, dec_b2.reshape(1, HID),
        dec_w3, dec_b3.reshape(1, HID), dec_w4, dec_b4.reshape(1, N),
        gat_W.reshape(NHEADS, NHID), gat_a.reshape(NHEADS, 2 * NHID),
        out_W, out_a,
    )
    n_dev = min(len(jax.devices()), B)
    mesh = Mesh(np.array(jax.devices()[:n_dev]), ("d",))
    f = shard_map(
        _run, mesh=mesh,
        in_specs=(P("d", None),) + (P(),) * len(rest),
        out_specs=(P("d", None), P("d", None)),
        check_rep=False,
    )
    return f(x, *rest)


# speedup vs baseline: 51.2601x; 51.2601x over previous
import jax
import jax.numpy as jnp
from jax.experimental import pallas as pl

N = 1024
B = 2
HID = 1024
LAT = 512
NHEADS = 4
NHID = 64
ALPHA = 0.2
_NEG_LOG2E = -1.4426950408889634


def _elu(v):
    return jnp.where(v > 0, v, jnp.exp(v) - 1.0)


def _att_nd(adj, colv, rowv, c1, c2):
    urow = rowv * c1
    vrow = rowv * c2
    smin = jnp.min(urow, axis=1, keepdims=True) + jnp.min(vrow, axis=1, keepdims=True)
    smax = jnp.max(urow, axis=1, keepdims=True) + jnp.max(vrow, axis=1, keepdims=True)
    sep_pos = smin[0, 0] > 0.0
    sep_neg = smax[0, 0] < 0.0
    sigma = jnp.where(sep_pos, 1.0, ALPHA)

    def _sep(_):
        q = jnp.exp2(vrow * (sigma * _NEG_LOG2E))
        den = jnp.sum(adj * q, axis=1, keepdims=True)
        num = jnp.sum(adj * (q * rowv), axis=1, keepdims=True)
        return num, den

    def _full(_):
        sp = colv * (c1 * _NEG_LOG2E) + vrow * _NEG_LOG2E
        e = jnp.exp2(jnp.minimum(sp, ALPHA * sp)) * adj
        den = jnp.sum(e, axis=1, keepdims=True)
        num = jnp.sum(e * rowv, axis=1, keepdims=True)
        return num, den

    return jax.lax.cond(jnp.logical_or(sep_pos, sep_neg), _sep, _full, 0)


def _deepim_body(x_ref, adj_ref, w1_ref, b1_ref, w2_ref, b2_ref, w3_ref, b3_ref,
                 d1_ref, db1_ref, d2_ref, db2_ref, d3_ref, db3_ref, d4_ref, db4_ref,
                 gw_ref, ga_ref, ow_ref, oa_ref, xhat_ref, yhat_ref):
    x = x_ref[...]
    adj = adj_ref[...]

    def _mm(a, w_ref):
        return jnp.dot(a, w_ref[...], preferred_element_type=jnp.float32)

    h = jax.nn.relu(_mm(x, w1_ref) + b1_ref[...])
    h = jax.nn.relu(_mm(h, w2_ref) + b2_ref[...])
    h = jax.nn.relu(_mm(h, w2_ref) + b2_ref[...])
    z = _mm(h, w3_ref) + b3_ref[...]
    h = jax.nn.relu(_mm(z, d1_ref) + db1_ref[...])
    h = jax.nn.relu(_mm(h, d2_ref) + db2_ref[...])
    h = jax.nn.relu(_mm(h, d3_ref) + db3_ref[...])
    x_hat = jax.nn.sigmoid(_mm(h, d4_ref) + db4_ref[...])
    xhat_ref[...] = x_hat

    gw = gw_ref[...]
    ga = ga_ref[...]
    ow = ow_ref[...]
    oa = oa_ref[...]

    for b in range(B):
        xn_row = x_hat[b:b + 1, :]
        xn_col = jnp.transpose(xn_row)

        blocks = []
        for i in range(NHEADS):
            gw_i = gw[i:i + 1, :]
            c1 = jnp.sum(gw_i * ga[i:i + 1, :NHID], axis=1, keepdims=True)
            c2 = jnp.sum(gw_i * ga[i:i + 1, NHID:], axis=1, keepdims=True)
            num, den = _att_nd(adj, xn_col, xn_row, c1, c2)
            p = num / den
            blocks.append(_elu(_elu(p * gw_i)))
        xh = jnp.concatenate(blocks, axis=1)

        h2 = jnp.dot(xh, ow, preferred_element_type=jnp.float32)
        h2_row = jnp.transpose(h2)
        num2, den2 = _att_nd(adj, h2, h2_row, oa[0:1, 0:1], oa[0:1, 1:2])
        y = _elu(num2 / den2)
        yhat_ref[b:b + 1, :] = jnp.transpose(y)


def kernel(x, adj, enc_w1, enc_b1, enc_w2, enc_b2, enc_w3, enc_b3,
           dec_w1, dec_b1, dec_w2, dec_b2, dec_w3, dec_b3, dec_w4, dec_b4,
           gat_W, gat_a, out_W, out_a):
    args = (
        x, adj,
        enc_w1, enc_b1.reshape(1, HID), enc_w2, enc_b2.reshape(1, HID),
        enc_w3, enc_b3.reshape(1, LAT),
        dec_w1, dec_b1.reshape(1, LAT), dec_w2, dec_b2.reshape(1, HID),
        dec_w3, dec_b3.reshape(1, HID), dec_w4, dec_b4.reshape(1, N),
        gat_W.reshape(NHEADS, NHID), gat_a.reshape(NHEADS, 2 * NHID),
        out_W, out_a,
    )
    x_hat, y_hat = pl.pallas_call(
        _deepim_body,
        out_shape=(
            jax.ShapeDtypeStruct((B, N), jnp.float32),
            jax.ShapeDtypeStruct((B, N), jnp.float32),
        ),
    )(*args)
    return x_hat, y_hat


# speedup vs baseline: 51.3378x; 1.0015x over previous
import jax
import jax.numpy as jnp
from jax.experimental import pallas as pl

N = 1024
B = 2
HID = 1024
LAT = 512
NHEADS = 4
NHID = 64
ALPHA = 0.2
_NEG_LOG2E = -1.4426950408889634


def _elu(v):
    return jnp.where(v > 0, v, jnp.exp(v) - 1.0)


def _att_nd(adj, colv, rowv, c1, c2):
    urow = rowv * c1
    vrow = rowv * c2
    smin = jnp.min(urow, axis=1, keepdims=True) + jnp.min(vrow, axis=1, keepdims=True)
    smax = jnp.max(urow, axis=1, keepdims=True) + jnp.max(vrow, axis=1, keepdims=True)
    sep_pos = smin[0, 0] > 0.0
    sep_neg = smax[0, 0] < 0.0
    sigma = jnp.where(sep_pos, 1.0, ALPHA)

    def _sep(_):
        q = jnp.exp2(vrow * (sigma * _NEG_LOG2E))
        den = jnp.sum(adj * q, axis=1, keepdims=True)
        num = jnp.sum(adj * (q * rowv), axis=1, keepdims=True)
        return num, den

    def _full(_):
        sp = colv * (c1 * _NEG_LOG2E) + vrow * _NEG_LOG2E
        e = jnp.exp2(jnp.minimum(sp, ALPHA * sp)) * adj
        den = jnp.sum(e, axis=1, keepdims=True)
        num = jnp.sum(e * rowv, axis=1, keepdims=True)
        return num, den

    return jax.lax.cond(jnp.logical_or(sep_pos, sep_neg), _sep, _full, 0)


def _deepim_body(x_ref, adj_ref, w1_ref, b1_ref, w2_ref, b2_ref, w3_ref, b3_ref,
                 d1_ref, db1_ref, d2_ref, db2_ref, d3_ref, db3_ref, d4_ref, db4_ref,
                 gw_ref, ga_ref, ow_ref, oa_ref, xhat_ref, yhat_ref):
    x = x_ref[...]
    adj = adj_ref[...]

    def _mm(a, w_ref):
        return jnp.dot(a, w_ref[...], preferred_element_type=jnp.float32,
                       precision=jax.lax.Precision.DEFAULT)

    h = jax.nn.relu(_mm(x, w1_ref) + b1_ref[...])
    h = jax.nn.relu(_mm(h, w2_ref) + b2_ref[...])
    h = jax.nn.relu(_mm(h, w2_ref) + b2_ref[...])
    z = _mm(h, w3_ref) + b3_ref[...]
    h = jax.nn.relu(_mm(z, d1_ref) + db1_ref[...])
    h = jax.nn.relu(_mm(h, d2_ref) + db2_ref[...])
    h = jax.nn.relu(_mm(h, d3_ref) + db3_ref[...])
    x_hat = jax.nn.sigmoid(_mm(h, d4_ref) + db4_ref[...])
    xhat_ref[...] = x_hat

    gw = gw_ref[...]
    ga = ga_ref[...]
    ow = ow_ref[...]
    oa = oa_ref[...]

    for b in range(B):
        xn_row = x_hat[b:b + 1, :]
        xn_col = jnp.transpose(xn_row)

        blocks = []
        for i in range(NHEADS):
            gw_i = gw[i:i + 1, :]
            c1 = jnp.sum(gw_i * ga[i:i + 1, :NHID], axis=1, keepdims=True)
            c2 = jnp.sum(gw_i * ga[i:i + 1, NHID:], axis=1, keepdims=True)
            num, den = _att_nd(adj, xn_col, xn_row, c1, c2)
            p = num / den
            blocks.append(_elu(_elu(p * gw_i)))
        xh = jnp.concatenate(blocks, axis=1)

        h2 = jnp.dot(xh, ow, preferred_element_type=jnp.float32)
        h2_row = jnp.transpose(h2)
        num2, den2 = _att_nd(adj, h2, h2_row, oa[0:1, 0:1], oa[0:1, 1:2])
        y = _elu(num2 / den2)
        yhat_ref[b:b + 1, :] = jnp.transpose(y)


def kernel(x, adj, enc_w1, enc_b1, enc_w2, enc_b2, enc_w3, enc_b3,
           dec_w1, dec_b1, dec_w2, dec_b2, dec_w3, dec_b3, dec_w4, dec_b4,
           gat_W, gat_a, out_W, out_a):
    args = (
        x, adj,
        enc_w1, enc_b1.reshape(1, HID), enc_w2, enc_b2.reshape(1, HID),
        enc_w3, enc_b3.reshape(1, LAT),
        dec_w1, dec_b1.reshape(1, LAT), dec_w2, dec_b2.reshape(1, HID),
        dec_w3, dec_b3.reshape(1, HID), dec_w4, dec_b4.reshape(1, N),
        gat_W.reshape(NHEADS, NHID), gat_a.reshape(NHEADS, 2 * NHID),
        out_W, out_a,
    )
    x_hat, y_hat = pl.pallas_call(
        _deepim_body,
        out_shape=(
            jax.ShapeDtypeStruct((B, N), jnp.float32),
            jax.ShapeDtypeStruct((B, N), jnp.float32),
        ),
    )(*args)
    return x_hat, y_hat


# speedup vs baseline: 52.8991x; 1.0304x over previous
import jax
import jax.numpy as jnp
from jax.experimental import pallas as pl
from jax.experimental.pallas import tpu as pltpu

N = 1024
B = 2
HID = 1024
LAT = 512
NHEADS = 4
NHID = 64
ALPHA = 0.2
_NEG_LOG2E = -1.4426950408889634


def _elu(v):
    return jnp.where(v > 0, v, jnp.exp(v) - 1.0)


def _att_nd(adj, colv, rowv, c1, c2):
    urow = rowv * c1
    vrow = rowv * c2
    smin = jnp.min(urow, axis=1, keepdims=True) + jnp.min(vrow, axis=1, keepdims=True)
    smax = jnp.max(urow, axis=1, keepdims=True) + jnp.max(vrow, axis=1, keepdims=True)
    sep_pos = smin[0, 0] > 0.0
    sep_neg = smax[0, 0] < 0.0
    sigma = jnp.where(sep_pos, 1.0, ALPHA)

    def _sep(_):
        q = jnp.exp2(vrow * (sigma * _NEG_LOG2E))
        den = jnp.sum(adj * q, axis=1, keepdims=True)
        num = jnp.sum(adj * (q * rowv), axis=1, keepdims=True)
        return num, den

    def _full(_):
        sp = colv * (c1 * _NEG_LOG2E) + vrow * _NEG_LOG2E
        e = jnp.exp2(jnp.minimum(sp, ALPHA * sp)) * adj
        den = jnp.sum(e, axis=1, keepdims=True)
        num = jnp.sum(e * rowv, axis=1, keepdims=True)
        return num, den

    return jax.lax.cond(jnp.logical_or(sep_pos, sep_neg), _sep, _full, 0)


def _deepim_body(x_ref, adjh_ref, w1h_ref, b1_ref, w2h_ref, b2_ref, w3h_ref, b3_ref,
                 d1h_ref, db1_ref, d2h_ref, db2_ref, d3h_ref, db3_ref, d4h_ref, db4_ref,
                 gw_ref, ga_ref, ow_ref, oa_ref, xhat_ref, yhat_ref,
                 w1_ref, w2_ref, w3_ref, d1_ref, d2_ref, d3_ref, d4_ref, adj_ref,
                 *sems):
    hbm = (w1h_ref, w2h_ref, w3h_ref, d1h_ref, d2h_ref, d3h_ref, d4h_ref, adjh_ref)
    vmem = (w1_ref, w2_ref, w3_ref, d1_ref, d2_ref, d3_ref, d4_ref, adj_ref)
    copies = [pltpu.make_async_copy(s, d, sem) for s, d, sem in zip(hbm, vmem, sems)]
    for c in copies:
        c.start()

    x = x_ref[...]

    def _mm(a, w_ref):
        return jnp.dot(a, w_ref[...], preferred_element_type=jnp.float32,
                       precision=jax.lax.Precision.DEFAULT)

    copies[0].wait()
    h = jax.nn.relu(_mm(x, w1_ref) + b1_ref[...])
    copies[1].wait()
    h = jax.nn.relu(_mm(h, w2_ref) + b2_ref[...])
    h = jax.nn.relu(_mm(h, w2_ref) + b2_ref[...])
    copies[2].wait()
    z = _mm(h, w3_ref) + b3_ref[...]
    copies[3].wait()
    h = jax.nn.relu(_mm(z, d1_ref) + db1_ref[...])
    copies[4].wait()
    h = jax.nn.relu(_mm(h, d2_ref) + db2_ref[...])
    copies[5].wait()
    h = jax.nn.relu(_mm(h, d3_ref) + db3_ref[...])
    copies[6].wait()
    x_hat = jax.nn.sigmoid(_mm(h, d4_ref) + db4_ref[...])
    xhat_ref[...] = x_hat

    copies[7].wait()
    adj = adj_ref[...]

    gw = gw_ref[...]
    ga = ga_ref[...]
    ow = ow_ref[...]
    oa = oa_ref[...]

    for b in range(B):
        xn_row = x_hat[b:b + 1, :]
        xn_col = jnp.transpose(xn_row)

        blocks = []
        for i in range(NHEADS):
            gw_i = gw[i:i + 1, :]
            c1 = jnp.sum(gw_i * ga[i:i + 1, :NHID], axis=1, keepdims=True)
            c2 = jnp.sum(gw_i * ga[i:i + 1, NHID:], axis=1, keepdims=True)
            num, den = _att_nd(adj, xn_col, xn_row, c1, c2)
            p = num / den
            blocks.append(_elu(_elu(p * gw_i)))
        xh = jnp.concatenate(blocks, axis=1)

        h2 = jnp.dot(xh, ow, preferred_element_type=jnp.float32)
        h2_row = jnp.transpose(h2)
        num2, den2 = _att_nd(adj, h2, h2_row, oa[0:1, 0:1], oa[0:1, 1:2])
        y = _elu(num2 / den2)
        yhat_ref[b:b + 1, :] = jnp.transpose(y)


def kernel(x, adj, enc_w1, enc_b1, enc_w2, enc_b2, enc_w3, enc_b3,
           dec_w1, dec_b1, dec_w2, dec_b2, dec_w3, dec_b3, dec_w4, dec_b4,
           gat_W, gat_a, out_W, out_a):
    args = (
        x, adj,
        enc_w1, enc_b1.reshape(1, HID), enc_w2, enc_b2.reshape(1, HID),
        enc_w3, enc_b3.reshape(1, LAT),
        dec_w1, dec_b1.reshape(1, LAT), dec_w2, dec_b2.reshape(1, HID),
        dec_w3, dec_b3.reshape(1, HID), dec_w4, dec_b4.reshape(1, N),
        gat_W.reshape(NHEADS, NHID), gat_a.reshape(NHEADS, 2 * NHID),
        out_W, out_a,
    )
    _any = pl.MemorySpace.ANY
    _vm = pltpu.MemorySpace.VMEM
    spaces = [_vm, _any, _any, _vm, _any, _vm, _any, _vm, _any, _vm, _any,
              _vm, _any, _vm, _any, _vm, _vm, _vm, _vm, _vm]
    x_hat, y_hat = pl.pallas_call(
        _deepim_body,
        in_specs=[pl.BlockSpec(memory_space=s) for s in spaces],
        out_shape=(
            jax.ShapeDtypeStruct((B, N), jnp.float32),
            jax.ShapeDtypeStruct((B, N), jnp.float32),
        ),
        scratch_shapes=[
            pltpu.VMEM((N, HID), jnp.float32),
            pltpu.VMEM((HID, HID), jnp.float32),
            pltpu.VMEM((HID, LAT), jnp.float32),
            pltpu.VMEM((LAT, LAT), jnp.float32),
            pltpu.VMEM((LAT, HID), jnp.float32),
            pltpu.VMEM((HID, HID), jnp.float32),
            pltpu.VMEM((HID, N), jnp.float32),
            pltpu.VMEM((N, N), jnp.float32),
        ] + [pltpu.SemaphoreType.DMA] * 8,
    )(*args)
    return x_hat, y_hat


# speedup vs baseline: 52.9735x; 1.0014x over previous
import jax
import jax.numpy as jnp
from jax.experimental import pallas as pl
from jax.experimental.pallas import tpu as pltpu

N = 1024
B = 2
HID = 1024
LAT = 512
NHEADS = 4
NHID = 64
ALPHA = 0.2
_NEG_LOG2E = -1.4426950408889634


def _elu(v):
    return jnp.where(v > 0, v, jnp.exp(v) - 1.0)


def _att_nd(adj, colv, rowv, c1, c2):
    urow = rowv * c1
    vrow = rowv * c2
    smin = jnp.min(urow, axis=1, keepdims=True) + jnp.min(vrow, axis=1, keepdims=True)
    smax = jnp.max(urow, axis=1, keepdims=True) + jnp.max(vrow, axis=1, keepdims=True)
    sep_pos = smin[0, 0] > 0.0
    sep_neg = smax[0, 0] < 0.0
    sigma = jnp.where(sep_pos, 1.0, ALPHA)

    def _sep(_):
        q = jnp.exp2(vrow * (sigma * _NEG_LOG2E))
        den = jnp.sum(adj * q, axis=1, keepdims=True)
        num = jnp.sum(adj * (q * rowv), axis=1, keepdims=True)
        return num, den

    def _full(_):
        sp = colv * (c1 * _NEG_LOG2E) + vrow * _NEG_LOG2E
        e = jnp.exp2(jnp.minimum(sp, ALPHA * sp)) * adj
        den = jnp.sum(e, axis=1, keepdims=True)
        num = jnp.sum(e * rowv, axis=1, keepdims=True)
        return num, den

    return jax.lax.cond(jnp.logical_or(sep_pos, sep_neg), _sep, _full, 0)


def _deepim_body(x_ref, adjh_ref, w1h_ref, b1_ref, w2h_ref, b2_ref, w3h_ref, b3_ref,
                 d1h_ref, db1_ref, d2h_ref, db2_ref, d3h_ref, db3_ref, d4h_ref, db4_ref,
                 gw_ref, ga_ref, ow_ref, oa_ref, xhat_ref, yhat_ref,
                 w1_ref, w2_ref, w3_ref, d1_ref, d2_ref, d3_ref, d4_ref, adj_ref,
                 *sems):
    hbm = (w1h_ref, w2h_ref, w3h_ref, d1h_ref, d2h_ref, d3h_ref, d4h_ref, adjh_ref)
    vmem = (w1_ref, w2_ref, w3_ref, d1_ref, d2_ref, d3_ref, d4_ref, adj_ref)
    copies = [pltpu.make_async_copy(s, d, sem) for s, d, sem in zip(hbm, vmem, sems)]
    for c in copies:
        c.start()

    x = x_ref[...]

    def _mm(a, w_ref):
        return jnp.dot(a, w_ref[...], preferred_element_type=jnp.float32,
                       precision=jax.lax.Precision.DEFAULT)

    copies[0].wait()
    h = jax.nn.relu(_mm(x, w1_ref) + b1_ref[...])
    copies[1].wait()
    h = jax.nn.relu(_mm(h, w2_ref) + b2_ref[...])
    h = jax.nn.relu(_mm(h, w2_ref) + b2_ref[...])
    copies[2].wait()
    z = _mm(h, w3_ref) + b3_ref[...]
    copies[3].wait()
    h = jax.nn.relu(_mm(z, d1_ref) + db1_ref[...])
    copies[4].wait()
    h = jax.nn.relu(_mm(h, d2_ref) + db2_ref[...])
    copies[5].wait()
    h = jax.nn.relu(_mm(h, d3_ref) + db3_ref[...])
    copies[6].wait()
    x_hat = jax.nn.sigmoid(_mm(h, d4_ref) + db4_ref[...])
    xhat_ref[...] = x_hat

    copies[7].wait()
    adj = adj_ref[...]

    gw = gw_ref[...]
    ga = ga_ref[...]
    ow = ow_ref[...]
    oa = oa_ref[...]

    for b in range(B):
        xn_row = x_hat[b:b + 1, :]
        xn_col = jnp.transpose(xn_row)

        blocks = []
        for i in range(NHEADS):
            gw_i = gw[i:i + 1, :]
            c1 = jnp.sum(gw_i * ga[i:i + 1, :NHID], axis=1, keepdims=True)
            c2 = jnp.sum(gw_i * ga[i:i + 1, NHID:], axis=1, keepdims=True)
            num, den = _att_nd(adj, xn_col, xn_row, c1, c2)
            p = num / den
            blocks.append(_elu(_elu(p * gw_i)))
        xh = jnp.concatenate(blocks, axis=1)

        h2 = jnp.dot(xh, ow, preferred_element_type=jnp.float32,
                     precision=jax.lax.Precision.DEFAULT)
        h2_row = jnp.transpose(h2)
        num2, den2 = _att_nd(adj, h2, h2_row, oa[0:1, 0:1], oa[0:1, 1:2])
        y = _elu(num2 / den2)
        yhat_ref[b:b + 1, :] = jnp.transpose(y)


def kernel(x, adj, enc_w1, enc_b1, enc_w2, enc_b2, enc_w3, enc_b3,
           dec_w1, dec_b1, dec_w2, dec_b2, dec_w3, dec_b3, dec_w4, dec_b4,
           gat_W, gat_a, out_W, out_a):
    args = (
        x, adj,
        enc_w1, enc_b1.reshape(1, HID), enc_w2, enc_b2.reshape(1, HID),
        enc_w3, enc_b3.reshape(1, LAT),
        dec_w1, dec_b1.reshape(1, LAT), dec_w2, dec_b2.reshape(1, HID),
        dec_w3, dec_b3.reshape(1, HID), dec_w4, dec_b4.reshape(1, N),
        gat_W.reshape(NHEADS, NHID), gat_a.reshape(NHEADS, 2 * NHID),
        out_W, out_a,
    )
    _any = pl.MemorySpace.ANY
    _vm = pltpu.MemorySpace.VMEM
    spaces = [_vm, _any, _any, _vm, _any, _vm, _any, _vm, _any, _vm, _any,
              _vm, _any, _vm, _any, _vm, _vm, _vm, _vm, _vm]
    x_hat, y_hat = pl.pallas_call(
        _deepim_body,
        in_specs=[pl.BlockSpec(memory_space=s) for s in spaces],
        out_shape=(
            jax.ShapeDtypeStruct((B, N), jnp.float32),
            jax.ShapeDtypeStruct((B, N), jnp.float32),
        ),
        scratch_shapes=[
            pltpu.VMEM((N, HID), jnp.float32),
            pltpu.VMEM((HID, HID), jnp.float32),
            pltpu.VMEM((HID, LAT), jnp.float32),
            pltpu.VMEM((LAT, LAT), jnp.float32),
            pltpu.VMEM((LAT, HID), jnp.float32),
            pltpu.VMEM((HID, HID), jnp.float32),
            pltpu.VMEM((HID, N), jnp.float32),
            pltpu.VMEM((N, N), jnp.float32),
        ] + [pltpu.SemaphoreType.DMA] * 8,
    )(*args)
    return x_hat, y_hat
